# Initial kernel scaffold; baseline (speedup 1.0000x reference)
import jax
import jax.numpy as jnp
from jax.experimental import pallas as pl


def kernel(i, encoding):
    raise NotImplementedError("write your pallas kernel here")


# speedup vs baseline: 1.4925x; 1.4925x over previous
import functools

import jax
import jax.numpy as jnp
from jax import lax
from jax.experimental import pallas as pl
from jax.experimental.pallas import tpu as pltpu
from jax.experimental.pallas import tpu_sc as plsc

_LENGTH = 8192
_CHANNELS = 1024
_SEQ = 8192
_BATCH = 4

_NC = 2
_NS = 16
_NW = _NC * _NS
_B = _SEQ * _BATCH
_BPW = _B // _NW
_C = 32
_G = _BPW // _C

_mesh = plsc.VectorSubcoreMesh(core_axis_name="c", subcore_axis_name="s")


@functools.partial(
    pl.kernel,
    out_type=jax.ShapeDtypeStruct((_B, _CHANNELS), jnp.float32),
    mesh=_mesh,
    scratch_types=[
        pltpu.VMEM((_G, _C), jnp.int32),
        pltpu.VMEM((_C, _CHANNELS), jnp.float32),
        pltpu.SemaphoreType.DMA,
    ],
)
def _gather_rows(idx_hbm, table_hbm, out_hbm, idx_v, rows_v, sem):
    wid = lax.axis_index("s") * _NC + lax.axis_index("c")
    base = wid * _BPW
    pltpu.sync_copy(idx_hbm.at[wid], idx_v)

    def body(g, carry):
        pltpu.async_copy(table_hbm.at[idx_v.at[g]], rows_v, sem).wait()
        pltpu.sync_copy(rows_v, out_hbm.at[pl.ds(base + g * _C, _C)])
        return carry

    lax.fori_loop(0, _G, body, 0)


def kernel(i, encoding):
    idx = i.astype(jnp.int32).reshape(_NW, _G, _C)
    out = _gather_rows(idx, encoding)
    return out.reshape(_SEQ, _BATCH, _CHANNELS)


# speedup vs baseline: 1.6339x; 1.0948x over previous
import functools

import jax
import jax.numpy as jnp
from jax import lax
from jax.experimental import pallas as pl
from jax.experimental.pallas import tpu as pltpu
from jax.experimental.pallas import tpu_sc as plsc

_LENGTH = 8192
_CHANNELS = 1024
_SEQ = 8192
_BATCH = 4

_NC = 2
_NS = 16
_NW = _NC * _NS
_B = _SEQ * _BATCH
_BPW = _B // _NW
_C = 32
_G = _BPW // _C

_mesh = plsc.VectorSubcoreMesh(core_axis_name="c", subcore_axis_name="s")


@functools.partial(
    pl.kernel,
    out_type=jax.ShapeDtypeStruct((_B, _CHANNELS), jnp.float32),
    mesh=_mesh,
    scratch_types=[
        pltpu.VMEM((_G, _C), jnp.int32),
        pltpu.VMEM((_C, _CHANNELS), jnp.float32),
        pltpu.VMEM((_C, _CHANNELS), jnp.float32),
        pltpu.SemaphoreType.DMA,
        pltpu.SemaphoreType.DMA,
        pltpu.SemaphoreType.DMA,
        pltpu.SemaphoreType.DMA,
    ],
)
def _gather_rows(idx_hbm, table_hbm, out_hbm, idx_v, buf_a, buf_b,
                 gsem_a, gsem_b, ssem_a, ssem_b):
    wid = lax.axis_index("s") * _NC + lax.axis_index("c")
    base = wid * _BPW
    pltpu.sync_copy(idx_hbm.at[wid], idx_v)

    def start_gather(g, buf, sem):
        pltpu.async_copy(table_hbm.at[idx_v.at[g]], buf, sem)

    def wait_gather(g, buf, sem):
        pltpu.make_async_copy(table_hbm.at[idx_v.at[g]], buf, sem).wait()

    def start_scatter(g, buf, sem):
        pltpu.async_copy(buf, out_hbm.at[pl.ds(base + g * _C, _C)], sem)

    def wait_scatter(g, buf, sem):
        pltpu.make_async_copy(buf, out_hbm.at[pl.ds(base + g * _C, _C)],
                              sem).wait()

    start_gather(0, buf_a, gsem_a)
    half = _G // 2

    def body(h, carry):
        g0 = 2 * h
        g1 = g0 + 1

        @pl.when(h > 0)
        def _():
            wait_scatter(g0 - 1, buf_b, ssem_b)

        start_gather(g1, buf_b, gsem_b)
        wait_gather(g0, buf_a, gsem_a)
        start_scatter(g0, buf_a, ssem_a)
        wait_scatter(g0, buf_a, ssem_a)

        @pl.when(h < half - 1)
        def _():
            start_gather(g0 + 2, buf_a, gsem_a)

        wait_gather(g1, buf_b, gsem_b)
        start_scatter(g1, buf_b, ssem_b)
        return carry

    lax.fori_loop(0, half, body, 0)
    wait_scatter(_G - 1, buf_b, ssem_b)


def kernel(i, encoding):
    idx = i.astype(jnp.int32).reshape(_NW, _G, _C)
    out = _gather_rows(idx, encoding)
    return out.reshape(_SEQ, _BATCH, _CHANNELS)
